# Initial kernel scaffold; baseline (speedup 1.0000x reference)
import jax
import jax.numpy as jnp
from jax.experimental import pallas as pl


def kernel(features, segment_target, rand_global_inds, W_feat, b_feat, W_key, b_key, W_query, b_query):
    raise NotImplementedError("write your pallas kernel here")


# speedup vs baseline: 230.1894x; 230.1894x over previous
import functools

import numpy as np
import jax
import jax.numpy as jnp
from jax import lax
from jax.experimental import pallas as pl
from jax.experimental.pallas import tpu as pltpu
from jax.experimental.pallas import tpu_sc as plsc

_B = 2
_IN_DIM = 256
_C = 32
_H = _W = 64
_N = _H * _W
_KWIN = 25
_S = 1024
_NLOC = _KWIN * _KWIN
_NRAND = _S - _NLOC
_ROWS = _B * _N
_INV_SQRT_C = float(_C) ** -0.5

_NUM_WORKERS = 32
_ROWS_PER_W = _ROWS // _NUM_WORKERS
_RCHUNK = 8
_NCHUNK = _ROWS_PER_W // _RCHUNK
_NGROUP = _S // 16


def _local_window_inds():
    half = (_KWIN - 1) // 2
    idx = np.arange(_N, dtype=np.int64).reshape(_H, _W)
    padded = np.zeros((_H + 2 * half, _W + 2 * half), dtype=np.int64)
    padded[half:half + _H, half:half + _W] = idx
    win = np.lib.stride_tricks.sliding_window_view(padded, (_KWIN, _KWIN))
    return win.reshape(_N, _NLOC).astype(np.int32)


_LOCAL_INDS = _local_window_inds()


def _proj_body(x_ref, wq_ref, wk_ref, qb_ref, kb_ref, q_ref, kt_ref):
    x = x_ref[0]
    q = lax.dot_general(x, wq_ref[...], (((0,), (1,)), ((), ())),
                        preferred_element_type=jnp.float32,
                        precision=lax.Precision.HIGHEST)
    q_ref[0] = q + qb_ref[...]
    kt = lax.dot_general(wk_ref[...], x, (((1,), (0,)), ((), ())),
                         preferred_element_type=jnp.float32,
                         precision=lax.Precision.HIGHEST)
    kt_ref[0] = kt + kb_ref[...]


def _projections(x, wqc, wkc, qbias, kbias):
    bn = 2048
    grid = (_B, _N // bn)
    return pl.pallas_call(
        _proj_body,
        grid=grid,
        in_specs=[
            pl.BlockSpec((1, _IN_DIM, bn), lambda b, j: (b, 0, j)),
            pl.BlockSpec((_C, _IN_DIM), lambda b, j: (0, 0)),
            pl.BlockSpec((_C, _IN_DIM), lambda b, j: (0, 0)),
            pl.BlockSpec((1, _C), lambda b, j: (0, 0)),
            pl.BlockSpec((_C, 1), lambda b, j: (0, 0)),
        ],
        out_specs=[
            pl.BlockSpec((1, bn, _C), lambda b, j: (b, j, 0)),
            pl.BlockSpec((1, _C, bn), lambda b, j: (b, 0, j)),
        ],
        out_shape=[
            jax.ShapeDtypeStruct((_B, _N, _C), jnp.float32),
            jax.ShapeDtypeStruct((_B, _C, _N), jnp.float32),
        ],
    )(x, wqc, wkc, qbias, kbias)


def _aff_body(q_ref, kt_ref, a_ref):
    a = lax.dot_general(q_ref[0], kt_ref[0], (((1,), (0,)), ((), ())),
                        preferred_element_type=jnp.float32,
                        precision=lax.Precision.HIGHEST)
    a_ref[0] = a * _INV_SQRT_C


def _affinity(q, kt):
    bm, bn = 512, 2048
    grid = (_B, _N // bm, _N // bn)
    return pl.pallas_call(
        _aff_body,
        grid=grid,
        in_specs=[
            pl.BlockSpec((1, bm, _C), lambda b, i, j: (b, i, 0)),
            pl.BlockSpec((1, _C, bn), lambda b, i, j: (b, 0, j)),
        ],
        out_specs=pl.BlockSpec((1, bm, bn), lambda b, i, j: (b, i, j)),
        out_shape=jax.ShapeDtypeStruct((_B, _N, _N), jnp.float32),
    )(q, kt)


def _sc_body(a_hbm, idx_hbm, seg_hbm, outl_hbm, outs_hbm,
             a_v, idx_v, seg_v, outl_v, outs_v):
    wid = lax.axis_index("s") * 2 + lax.axis_index("c")
    batch = wid // 16
    pltpu.sync_copy(seg_hbm.at[pl.ds(batch * _N, _N)], seg_v)

    def chunk_body(ci, _):
        base = wid * _ROWS_PER_W + ci * _RCHUNK
        pltpu.sync_copy(a_hbm.at[pl.ds(base * _N, _RCHUNK * _N)], a_v)
        pltpu.sync_copy(idx_hbm.at[pl.ds(base * _S, _RCHUNK * _S)], idx_v)

        def row_body(r, _):
            arow = r * _N

            def grp_body(g, _):
                off = r * _S + g * 16
                cols = idx_v[pl.ds(off, 16)]
                outl_v[pl.ds(off, 16)] = plsc.load_gather(a_v, [arow + cols])
                outs_v[pl.ds(off, 16)] = plsc.load_gather(seg_v, [cols])
                return 0

            lax.fori_loop(0, _NGROUP, grp_body, 0, unroll=4)
            return 0

        lax.fori_loop(0, _RCHUNK, row_body, 0)
        pltpu.sync_copy(outl_v, outl_hbm.at[pl.ds(base * _S, _RCHUNK * _S)])
        pltpu.sync_copy(outs_v, outs_hbm.at[pl.ds(base * _S, _RCHUNK * _S)])
        return 0

    lax.fori_loop(0, _NCHUNK, chunk_body, 0)


def _sc_gather(a_flat, sample_inds, seg_flat):
    fn = functools.partial(
        pl.kernel,
        mesh=plsc.VectorSubcoreMesh(core_axis_name="c", subcore_axis_name="s"),
        compiler_params=pltpu.CompilerParams(use_tc_tiling_on_sc=False,
                                             needs_layout_passes=False),
        out_type=[
            jax.ShapeDtypeStruct((_ROWS * _S,), jnp.float32),
            jax.ShapeDtypeStruct((_ROWS * _S,), jnp.int32),
        ],
        scratch_types=[
            pltpu.VMEM((_RCHUNK * _N,), jnp.float32),
            pltpu.VMEM((_RCHUNK * _S,), jnp.int32),
            pltpu.VMEM((_N,), jnp.int32),
            pltpu.VMEM((_RCHUNK * _S,), jnp.float32),
            pltpu.VMEM((_RCHUNK * _S,), jnp.int32),
        ],
    )(_sc_body)
    return fn(a_flat.reshape(_ROWS * _N), sample_inds.reshape(_ROWS * _S),
              seg_flat)


def _loss_body(l_ref, s_ref, seg_ref, out_ref, acc_ref):
    step = pl.program_id(0)
    nsteps = pl.num_programs(0)

    @pl.when(step == 0)
    def _():
        acc_ref[0] = 0.0
        acc_ref[1] = 0.0

    logits = l_ref[...]
    samples = s_ref[...]
    seg = seg_ref[...]
    mask = 1.0 - (seg == 0).astype(jnp.float32)
    targets = (samples == seg).astype(jnp.float32)
    maxes = jnp.max(logits, axis=-1, keepdims=True)
    x_exp = jnp.exp(logits - maxes)
    x_exp_w = x_exp * mask
    denom = jnp.sum(x_exp_w, axis=-1, keepdims=True) + 1e-9
    y_pred = x_exp_w / denom
    y_pred_log = jnp.log(jnp.clip(y_pred, 1e-8, None))
    t_sum = jnp.sum(targets, axis=-1, keepdims=True) + 1e-9
    y_true = targets / t_sum
    kl = jnp.where(y_true > 0,
                   y_true * (jnp.log(jnp.clip(y_true, 1e-30, None)) - y_pred_log),
                   0.0) * mask
    acc_ref[0] += jnp.sum(kl)
    acc_ref[1] += jnp.sum(mask)

    @pl.when(step == nsteps - 1)
    def _():
        out_ref[0, 0] = acc_ref[0] / (acc_ref[1] + 1e-9)


def _loss(logits_flat, samples, seg_col):
    rb = 512
    grid = (_ROWS // rb,)
    return pl.pallas_call(
        _loss_body,
        grid=grid,
        in_specs=[
            pl.BlockSpec((rb, _S), lambda i: (i, 0)),
            pl.BlockSpec((rb, _S), lambda i: (i, 0)),
            pl.BlockSpec((rb, 1), lambda i: (i, 0)),
        ],
        out_specs=pl.BlockSpec((1, 1), lambda i: (0, 0),
                               memory_space=pltpu.SMEM),
        out_shape=jax.ShapeDtypeStruct((1, 1), jnp.float32),
        scratch_shapes=[pltpu.SMEM((2,), jnp.float32)],
    )(logits_flat, samples, seg_col)


def kernel(features, segment_target, rand_global_inds,
           W_feat, b_feat, W_key, b_key, W_query, b_query):
    x = features.reshape(_B, _IN_DIM, _N)
    wqc = jnp.dot(W_query, W_feat, precision=lax.Precision.HIGHEST)
    wkc = jnp.dot(W_key, W_feat, precision=lax.Precision.HIGHEST)
    qbias = (jnp.dot(W_query, b_feat) + b_query).reshape(1, _C)
    kbias = (jnp.dot(W_key, b_feat) + b_key).reshape(_C, 1)

    q, kt = _projections(x, wqc, wkc, qbias, kbias)
    a = _affinity(q, kt).reshape(_ROWS, _N)

    local = jnp.broadcast_to(jnp.asarray(_LOCAL_INDS)[None], (_B, _N, _NLOC))
    sample_inds = jnp.concatenate(
        [local, rand_global_inds.astype(jnp.int32)], axis=-1).reshape(_ROWS, _S)
    seg_flat = segment_target.reshape(_ROWS).astype(jnp.int32)

    logits_flat, samples = _sc_gather(a, sample_inds, seg_flat)
    logits_flat = logits_flat.reshape(_ROWS, _S)
    samples = samples.reshape(_ROWS, _S)

    loss2d = _loss(logits_flat, samples, seg_flat.reshape(_ROWS, 1))
    loss = loss2d[0, 0]
    return (logits_flat.reshape(_B, _N, _S), loss)


# speedup vs baseline: 231.4319x; 1.0054x over previous
import functools

import numpy as np
import jax
import jax.numpy as jnp
from jax import lax
from jax.experimental import pallas as pl
from jax.experimental.pallas import tpu as pltpu
from jax.experimental.pallas import tpu_sc as plsc

_B = 2
_IN_DIM = 256
_C = 32
_H = _W = 64
_N = _H * _W
_KWIN = 25
_S = 1024
_NLOC = _KWIN * _KWIN
_NRAND = _S - _NLOC
_ROWS = _B * _N
_INV_SQRT_C = float(_C) ** -0.5

_NUM_WORKERS = 32
_ROWS_PER_W = _ROWS // _NUM_WORKERS
_RCHUNK = 8
_NCHUNK = _ROWS_PER_W // _RCHUNK
_NGROUP = _S // 16


def _local_window_inds():
    half = (_KWIN - 1) // 2
    idx = np.arange(_N, dtype=np.int64).reshape(_H, _W)
    padded = np.zeros((_H + 2 * half, _W + 2 * half), dtype=np.int64)
    padded[half:half + _H, half:half + _W] = idx
    win = np.lib.stride_tricks.sliding_window_view(padded, (_KWIN, _KWIN))
    return win.reshape(_N, _NLOC).astype(np.int32)


_LOCAL_INDS = _local_window_inds()


def _proj_body(x_ref, wq_ref, wk_ref, qb_ref, kb_ref, q_ref, kt_ref):
    x = x_ref[0]
    q = lax.dot_general(x, wq_ref[...], (((0,), (1,)), ((), ())),
                        preferred_element_type=jnp.float32,
                        precision=lax.Precision.HIGHEST)
    q_ref[0] = q + qb_ref[...]
    kt = lax.dot_general(wk_ref[...], x, (((1,), (0,)), ((), ())),
                         preferred_element_type=jnp.float32,
                         precision=lax.Precision.HIGHEST)
    kt_ref[0] = kt + kb_ref[...]


def _projections(x, wqc, wkc, qbias, kbias):
    bn = 2048
    grid = (_B, _N // bn)
    return pl.pallas_call(
        _proj_body,
        grid=grid,
        in_specs=[
            pl.BlockSpec((1, _IN_DIM, bn), lambda b, j: (b, 0, j)),
            pl.BlockSpec((_C, _IN_DIM), lambda b, j: (0, 0)),
            pl.BlockSpec((_C, _IN_DIM), lambda b, j: (0, 0)),
            pl.BlockSpec((1, _C), lambda b, j: (0, 0)),
            pl.BlockSpec((_C, 1), lambda b, j: (0, 0)),
        ],
        out_specs=[
            pl.BlockSpec((1, bn, _C), lambda b, j: (b, j, 0)),
            pl.BlockSpec((1, _C, bn), lambda b, j: (b, 0, j)),
        ],
        out_shape=[
            jax.ShapeDtypeStruct((_B, _N, _C), jnp.float32),
            jax.ShapeDtypeStruct((_B, _C, _N), jnp.float32),
        ],
    )(x, wqc, wkc, qbias, kbias)


def _aff_body(q_ref, kt_ref, a_ref):
    a = lax.dot_general(q_ref[0], kt_ref[0], (((1,), (0,)), ((), ())),
                        preferred_element_type=jnp.float32,
                        precision=lax.Precision.HIGHEST)
    a_ref[0] = a * _INV_SQRT_C


def _affinity(q, kt):
    bm, bn = 512, 2048
    grid = (_B, _N // bm, _N // bn)
    return pl.pallas_call(
        _aff_body,
        grid=grid,
        in_specs=[
            pl.BlockSpec((1, bm, _C), lambda b, i, j: (b, i, 0)),
            pl.BlockSpec((1, _C, bn), lambda b, i, j: (b, 0, j)),
        ],
        out_specs=pl.BlockSpec((1, bm, bn), lambda b, i, j: (b, i, j)),
        out_shape=jax.ShapeDtypeStruct((_B, _N, _N), jnp.float32),
    )(q, kt)


def _idx_body(loc_ref, rand_ref, out_ref):
    out_ref[:, 0:_NLOC] = loc_ref[...]
    out_ref[:, _NLOC:_S] = rand_ref[0]


def _assemble_inds(local_tab, rand):
    rb = 2048
    grid = (_B, _N // rb)
    return pl.pallas_call(
        _idx_body,
        grid=grid,
        in_specs=[
            pl.BlockSpec((rb, _NLOC), lambda b, i: (i, 0)),
            pl.BlockSpec((1, rb, _NRAND), lambda b, i: (b, i, 0)),
        ],
        out_specs=pl.BlockSpec((rb, _S), lambda b, i: (b * (_N // rb) + i, 0)),
        out_shape=jax.ShapeDtypeStruct((_ROWS, _S), jnp.int32),
    )(local_tab, rand)


def _sc_body(a_hbm, idx_hbm, seg_hbm, outl_hbm, outs_hbm,
             a_v, idx_v, seg_v, outl_v, outs_v):
    wid = lax.axis_index("s") * 2 + lax.axis_index("c")
    batch = wid // 16
    pltpu.sync_copy(seg_hbm.at[pl.ds(batch * _N, _N)], seg_v)

    def chunk_body(ci, _):
        base = wid * _ROWS_PER_W + ci * _RCHUNK
        pltpu.sync_copy(a_hbm.at[pl.ds(base * _N, _RCHUNK * _N)], a_v)
        pltpu.sync_copy(idx_hbm.at[pl.ds(base * _S, _RCHUNK * _S)], idx_v)

        def row_body(r, _):
            arow = r * _N

            def grp_body(g, _):
                off = r * _S + g * 16
                cols = idx_v[pl.ds(off, 16)]
                outl_v[pl.ds(off, 16)] = plsc.load_gather(a_v, [arow + cols])
                outs_v[pl.ds(off, 16)] = plsc.load_gather(seg_v, [cols])
                return 0

            lax.fori_loop(0, _NGROUP, grp_body, 0, unroll=4)
            return 0

        lax.fori_loop(0, _RCHUNK, row_body, 0)
        pltpu.sync_copy(outl_v, outl_hbm.at[pl.ds(base * _S, _RCHUNK * _S)])
        pltpu.sync_copy(outs_v, outs_hbm.at[pl.ds(base * _S, _RCHUNK * _S)])
        return 0

    lax.fori_loop(0, _NCHUNK, chunk_body, 0)


def _sc_gather(a_flat, sample_inds, seg_flat):
    fn = functools.partial(
        pl.kernel,
        mesh=plsc.VectorSubcoreMesh(core_axis_name="c", subcore_axis_name="s"),
        compiler_params=pltpu.CompilerParams(use_tc_tiling_on_sc=False,
                                             needs_layout_passes=False),
        out_type=[
            jax.ShapeDtypeStruct((_ROWS * _S,), jnp.float32),
            jax.ShapeDtypeStruct((_ROWS * _S,), jnp.int32),
        ],
        scratch_types=[
            pltpu.VMEM((_RCHUNK * _N,), jnp.float32),
            pltpu.VMEM((_RCHUNK * _S,), jnp.int32),
            pltpu.VMEM((_N,), jnp.int32),
            pltpu.VMEM((_RCHUNK * _S,), jnp.float32),
            pltpu.VMEM((_RCHUNK * _S,), jnp.int32),
        ],
    )(_sc_body)
    return fn(a_flat.reshape(_ROWS * _N), sample_inds.reshape(_ROWS * _S),
              seg_flat)


def _loss_body(l_ref, s_ref, seg_ref, out_ref, acc_ref):
    step = pl.program_id(0)
    nsteps = pl.num_programs(0)

    @pl.when(step == 0)
    def _():
        acc_ref[0] = 0.0
        acc_ref[1] = 0.0

    logits = l_ref[...]
    samples = s_ref[...]
    seg = seg_ref[...]
    mask = 1.0 - (seg == 0).astype(jnp.float32)
    targets = (samples == seg).astype(jnp.float32)
    maxes = jnp.max(logits, axis=-1, keepdims=True)
    x_exp = jnp.exp(logits - maxes)
    x_exp_w = x_exp * mask
    denom = jnp.sum(x_exp_w, axis=-1, keepdims=True) + 1e-9
    y_pred = x_exp_w / denom
    y_pred_log = jnp.log(jnp.clip(y_pred, 1e-8, None))
    t_sum = jnp.sum(targets, axis=-1, keepdims=True) + 1e-9
    y_true = targets / t_sum
    kl = jnp.where(y_true > 0,
                   y_true * (jnp.log(jnp.clip(y_true, 1e-30, None)) - y_pred_log),
                   0.0) * mask
    acc_ref[0] += jnp.sum(kl)
    acc_ref[1] += jnp.sum(mask)

    @pl.when(step == nsteps - 1)
    def _():
        out_ref[0, 0] = acc_ref[0] / (acc_ref[1] + 1e-9)


def _loss(logits_flat, samples, seg_col):
    rb = 512
    grid = (_ROWS // rb,)
    return pl.pallas_call(
        _loss_body,
        grid=grid,
        in_specs=[
            pl.BlockSpec((rb, _S), lambda i: (i, 0)),
            pl.BlockSpec((rb, _S), lambda i: (i, 0)),
            pl.BlockSpec((rb, 1), lambda i: (i, 0)),
        ],
        out_specs=pl.BlockSpec((1, 1), lambda i: (0, 0),
                               memory_space=pltpu.SMEM),
        out_shape=jax.ShapeDtypeStruct((1, 1), jnp.float32),
        scratch_shapes=[pltpu.SMEM((2,), jnp.float32)],
    )(logits_flat, samples, seg_col)


def kernel(features, segment_target, rand_global_inds,
           W_feat, b_feat, W_key, b_key, W_query, b_query):
    x = features.reshape(_B, _IN_DIM, _N)
    wqc = jnp.dot(W_query, W_feat, precision=lax.Precision.HIGHEST)
    wkc = jnp.dot(W_key, W_feat, precision=lax.Precision.HIGHEST)
    qbias = (jnp.dot(W_query, b_feat) + b_query).reshape(1, _C)
    kbias = (jnp.dot(W_key, b_feat) + b_key).reshape(_C, 1)

    q, kt = _projections(x, wqc, wkc, qbias, kbias)
    a = _affinity(q, kt).reshape(_ROWS, _N)

    sample_inds = _assemble_inds(jnp.asarray(_LOCAL_INDS),
                                 rand_global_inds.astype(jnp.int32))
    seg_flat = segment_target.reshape(_ROWS).astype(jnp.int32)

    logits_flat, samples = _sc_gather(a, sample_inds, seg_flat)
    logits_flat = logits_flat.reshape(_ROWS, _S)
    samples = samples.reshape(_ROWS, _S)

    loss2d = _loss(logits_flat, samples, seg_flat.reshape(_ROWS, 1))
    loss = loss2d[0, 0]
    return (logits_flat.reshape(_B, _N, _S), loss)


# speedup vs baseline: 292.9353x; 1.2658x over previous
import functools

import numpy as np
import jax
import jax.numpy as jnp
from jax import lax
from jax.experimental import pallas as pl
from jax.experimental.pallas import tpu as pltpu
from jax.experimental.pallas import tpu_sc as plsc

_B = 2
_IN_DIM = 256
_C = 32
_H = _W = 64
_N = _H * _W
_KWIN = 25
_S = 1024
_NLOC = _KWIN * _KWIN
_NRAND = _S - _NLOC
_ROWS = _B * _N
_INV_SQRT_C = float(_C) ** -0.5

_NUM_WORKERS = 32
_ROWS_PER_W = _ROWS // _NUM_WORKERS
_RCHUNK = 8
_NCHUNK = _ROWS_PER_W // _RCHUNK
_NGROUP = _S // 16


def _local_window_inds():
    half = (_KWIN - 1) // 2
    idx = np.arange(_N, dtype=np.int64).reshape(_H, _W)
    padded = np.zeros((_H + 2 * half, _W + 2 * half), dtype=np.int64)
    padded[half:half + _H, half:half + _W] = idx
    win = np.lib.stride_tricks.sliding_window_view(padded, (_KWIN, _KWIN))
    return win.reshape(_N, _NLOC).astype(np.int32)


_LOCAL_INDS = _local_window_inds()
_LOCAL_ADDR = ((np.arange(_N, dtype=np.int32)[:, None] & 7) << 12) + _LOCAL_INDS


def _proj_body(x_ref, wq_ref, wk_ref, qb_ref, kb_ref, q_ref, kt_ref):
    x = x_ref[0]
    q = lax.dot_general(x, wq_ref[...], (((0,), (1,)), ((), ())),
                        preferred_element_type=jnp.float32,
                        precision=lax.Precision.HIGHEST)
    q_ref[0] = q + qb_ref[...]
    kt = lax.dot_general(wk_ref[...], x, (((1,), (0,)), ((), ())),
                         preferred_element_type=jnp.float32,
                         precision=lax.Precision.HIGHEST)
    kt_ref[0] = kt + kb_ref[...]


def _projections(x, wqc, wkc, qbias, kbias):
    bn = 2048
    grid = (_B, _N // bn)
    return pl.pallas_call(
        _proj_body,
        grid=grid,
        in_specs=[
            pl.BlockSpec((1, _IN_DIM, bn), lambda b, j: (b, 0, j)),
            pl.BlockSpec((_C, _IN_DIM), lambda b, j: (0, 0)),
            pl.BlockSpec((_C, _IN_DIM), lambda b, j: (0, 0)),
            pl.BlockSpec((1, _C), lambda b, j: (0, 0)),
            pl.BlockSpec((_C, 1), lambda b, j: (0, 0)),
        ],
        out_specs=[
            pl.BlockSpec((1, bn, _C), lambda b, j: (b, j, 0)),
            pl.BlockSpec((1, _C, bn), lambda b, j: (b, 0, j)),
        ],
        out_shape=[
            jax.ShapeDtypeStruct((_B, _N, _C), jnp.float32),
            jax.ShapeDtypeStruct((_B, _C, _N), jnp.float32),
        ],
    )(x, wqc, wkc, qbias, kbias)


def _aff_body(q_ref, kt_ref, a_ref):
    a = lax.dot_general(q_ref[0], kt_ref[0], (((1,), (0,)), ((), ())),
                        preferred_element_type=jnp.float32,
                        precision=lax.Precision.HIGHEST)
    a_ref[0] = a * _INV_SQRT_C


def _affinity(q, kt):
    bm, bn = 512, 2048
    grid = (_B, _N // bm, _N // bn)
    return pl.pallas_call(
        _aff_body,
        grid=grid,
        in_specs=[
            pl.BlockSpec((1, bm, _C), lambda b, i, j: (b, i, 0)),
            pl.BlockSpec((1, _C, bn), lambda b, i, j: (b, 0, j)),
        ],
        out_specs=pl.BlockSpec((1, bm, bn), lambda b, i, j: (b, i, j)),
        out_shape=jax.ShapeDtypeStruct((_B, _N, _N), jnp.float32),
    )(q, kt)


def _idx_body(loc_ref, rand_ref, out_ref):
    rb = out_ref.shape[0]
    row = jax.lax.broadcasted_iota(jnp.int32, (rb, _NRAND), 0)
    out_ref[:, 0:_NLOC] = loc_ref[...]
    out_ref[:, _NLOC:_S] = ((row & 7) << 12) + rand_ref[0]


def _assemble_inds(local_tab, rand):
    rb = 2048
    grid = (_B, _N // rb)
    return pl.pallas_call(
        _idx_body,
        grid=grid,
        in_specs=[
            pl.BlockSpec((rb, _NLOC), lambda b, i: (i, 0)),
            pl.BlockSpec((1, rb, _NRAND), lambda b, i: (b, i, 0)),
        ],
        out_specs=pl.BlockSpec((rb, _S), lambda b, i: (b * (_N // rb) + i, 0)),
        out_shape=jax.ShapeDtypeStruct((_ROWS, _S), jnp.int32),
    )(local_tab, rand)


def _sc_body(a_hbm, idx_hbm, segtab_hbm, outl_hbm, outs_hbm,
             a_v, idx_v, segtab_v, outl_v, outs_v):
    wid = lax.axis_index("s") * 2 + lax.axis_index("c")
    batch = wid // 16
    pltpu.sync_copy(segtab_hbm.at[batch], segtab_v)

    def chunk_body(ci, _):
        base = wid * _ROWS_PER_W + ci * _RCHUNK
        pltpu.sync_copy(a_hbm.at[pl.ds(base, _RCHUNK)], a_v)
        pltpu.sync_copy(idx_hbm.at[pl.ds(base, _RCHUNK)], idx_v)

        def row_body(r, _):
            def grp_body(g, _):
                sl = pl.ds(g * 16, 16)
                addrs = idx_v[r, sl]
                hi = addrs >> 12
                lo = addrs & 4095
                outl_v[r, sl] = plsc.load_gather(a_v, [hi, lo])
                outs_v[r, sl] = plsc.load_gather(segtab_v, [hi, lo])
                return 0

            lax.fori_loop(0, _NGROUP, grp_body, 0, unroll=4)
            return 0

        lax.fori_loop(0, _RCHUNK, row_body, 0)
        pltpu.sync_copy(outl_v, outl_hbm.at[pl.ds(base, _RCHUNK)])
        pltpu.sync_copy(outs_v, outs_hbm.at[pl.ds(base, _RCHUNK)])
        return 0

    lax.fori_loop(0, _NCHUNK, chunk_body, 0)


def _sc_gather(a2, sample_addrs, segtab):
    fn = functools.partial(
        pl.kernel,
        mesh=plsc.VectorSubcoreMesh(core_axis_name="c", subcore_axis_name="s"),
        compiler_params=pltpu.CompilerParams(needs_layout_passes=False),
        out_type=[
            jax.ShapeDtypeStruct((_ROWS, _S), jnp.float32),
            jax.ShapeDtypeStruct((_ROWS, _S), jnp.int32),
        ],
        scratch_types=[
            pltpu.VMEM((_RCHUNK, _N), jnp.float32),
            pltpu.VMEM((_RCHUNK, _S), jnp.int32),
            pltpu.VMEM((_RCHUNK, _N), jnp.int32),
            pltpu.VMEM((_RCHUNK, _S), jnp.float32),
            pltpu.VMEM((_RCHUNK, _S), jnp.int32),
        ],
    )(_sc_body)
    return fn(a2, sample_addrs, segtab)


def _loss_body(l_ref, s_ref, seg_ref, out_ref, acc_ref):
    step = pl.program_id(0)
    nsteps = pl.num_programs(0)

    @pl.when(step == 0)
    def _():
        acc_ref[0] = 0.0
        acc_ref[1] = 0.0

    logits = l_ref[...]
    samples = s_ref[...]
    seg = seg_ref[...]
    mask = 1.0 - (seg == 0).astype(jnp.float32)
    targets = (samples == seg).astype(jnp.float32)
    maxes = jnp.max(logits, axis=-1, keepdims=True)
    x_exp = jnp.exp(logits - maxes)
    x_exp_w = x_exp * mask
    denom = jnp.sum(x_exp_w, axis=-1, keepdims=True) + 1e-9
    y_pred = x_exp_w / denom
    y_pred_log = jnp.log(jnp.clip(y_pred, 1e-8, None))
    t_sum = jnp.sum(targets, axis=-1, keepdims=True) + 1e-9
    y_true = targets / t_sum
    kl = jnp.where(y_true > 0,
                   y_true * (jnp.log(jnp.clip(y_true, 1e-30, None)) - y_pred_log),
                   0.0) * mask
    acc_ref[0] += jnp.sum(kl)
    acc_ref[1] += jnp.sum(mask)

    @pl.when(step == nsteps - 1)
    def _():
        out_ref[0, 0] = acc_ref[0] / (acc_ref[1] + 1e-9)


def _loss(logits_flat, samples, seg_col):
    rb = 512
    grid = (_ROWS // rb,)
    return pl.pallas_call(
        _loss_body,
        grid=grid,
        in_specs=[
            pl.BlockSpec((rb, _S), lambda i: (i, 0)),
            pl.BlockSpec((rb, _S), lambda i: (i, 0)),
            pl.BlockSpec((rb, 1), lambda i: (i, 0)),
        ],
        out_specs=pl.BlockSpec((1, 1), lambda i: (0, 0),
                               memory_space=pltpu.SMEM),
        out_shape=jax.ShapeDtypeStruct((1, 1), jnp.float32),
        scratch_shapes=[pltpu.SMEM((2,), jnp.float32)],
    )(logits_flat, samples, seg_col)


def kernel(features, segment_target, rand_global_inds,
           W_feat, b_feat, W_key, b_key, W_query, b_query):
    x = features.reshape(_B, _IN_DIM, _N)
    wqc = jnp.dot(W_query, W_feat, precision=lax.Precision.HIGHEST)
    wkc = jnp.dot(W_key, W_feat, precision=lax.Precision.HIGHEST)
    qbias = (jnp.dot(W_query, b_feat) + b_query).reshape(1, _C)
    kbias = (jnp.dot(W_key, b_feat) + b_key).reshape(_C, 1)

    q, kt = _projections(x, wqc, wkc, qbias, kbias)
    a = _affinity(q, kt).reshape(_ROWS, _N)

    sample_addrs = _assemble_inds(jnp.asarray(_LOCAL_ADDR),
                                  rand_global_inds.astype(jnp.int32))
    seg_flat = segment_target.reshape(_ROWS).astype(jnp.int32)
    segtab = jnp.broadcast_to(seg_flat.reshape(_B, 1, _N), (_B, _RCHUNK, _N))

    logits_flat, samples = _sc_gather(a, sample_addrs, segtab)

    loss2d = _loss(logits_flat, samples, seg_flat.reshape(_ROWS, 1))
    loss = loss2d[0, 0]
    return (logits_flat.reshape(_B, _N, _S), loss)


# speedup vs baseline: 364.7953x; 1.2453x over previous
import functools

import numpy as np
import jax
import jax.numpy as jnp
from jax import lax
from jax.experimental import pallas as pl
from jax.experimental.pallas import tpu as pltpu
from jax.experimental.pallas import tpu_sc as plsc

_B = 2
_IN_DIM = 256
_C = 32
_H = _W = 64
_N = _H * _W
_KWIN = 25
_S = 1024
_NLOC = _KWIN * _KWIN
_NRAND = _S - _NLOC
_ROWS = _B * _N
_INV_SQRT_C = float(_C) ** -0.5

_NUM_WORKERS = 32
_ROWS_PER_W = _ROWS // _NUM_WORKERS
_RCHUNK = 4
_NCHUNK = _ROWS_PER_W // _RCHUNK
_NGROUP = _S // 16
_SEGROWS = _RCHUNK


def _local_window_inds():
    half = (_KWIN - 1) // 2
    idx = np.arange(_N, dtype=np.int64).reshape(_H, _W)
    padded = np.zeros((_H + 2 * half, _W + 2 * half), dtype=np.int64)
    padded[half:half + _H, half:half + _W] = idx
    win = np.lib.stride_tricks.sliding_window_view(padded, (_KWIN, _KWIN))
    return win.reshape(_N, _NLOC).astype(np.int32)


_LOCAL_INDS = _local_window_inds()
_LOCAL_ADDR = (((np.arange(_N, dtype=np.int32)[:, None] % _RCHUNK) << 12)
               + _LOCAL_INDS)


def _proj_body(x_ref, wq_ref, wk_ref, qb_ref, kb_ref, q_ref, kt_ref):
    x = x_ref[0]
    q = lax.dot_general(x, wq_ref[...], (((0,), (1,)), ((), ())),
                        preferred_element_type=jnp.float32,
                        precision=lax.Precision.HIGHEST)
    q_ref[0] = q + qb_ref[...]
    kt = lax.dot_general(wk_ref[...], x, (((1,), (0,)), ((), ())),
                         preferred_element_type=jnp.float32,
                         precision=lax.Precision.HIGHEST)
    kt_ref[0] = kt + kb_ref[...]


def _projections(x, wqc, wkc, qbias, kbias):
    bn = 2048
    grid = (_B, _N // bn)
    return pl.pallas_call(
        _proj_body,
        grid=grid,
        in_specs=[
            pl.BlockSpec((1, _IN_DIM, bn), lambda b, j: (b, 0, j)),
            pl.BlockSpec((_C, _IN_DIM), lambda b, j: (0, 0)),
            pl.BlockSpec((_C, _IN_DIM), lambda b, j: (0, 0)),
            pl.BlockSpec((1, _C), lambda b, j: (0, 0)),
            pl.BlockSpec((_C, 1), lambda b, j: (0, 0)),
        ],
        out_specs=[
            pl.BlockSpec((1, bn, _C), lambda b, j: (b, j, 0)),
            pl.BlockSpec((1, _C, bn), lambda b, j: (b, 0, j)),
        ],
        out_shape=[
            jax.ShapeDtypeStruct((_B, _N, _C), jnp.float32),
            jax.ShapeDtypeStruct((_B, _C, _N), jnp.float32),
        ],
    )(x, wqc, wkc, qbias, kbias)


def _aff_body(q_ref, kt_ref, a_ref):
    a = lax.dot_general(q_ref[0], kt_ref[0], (((1,), (0,)), ((), ())),
                        preferred_element_type=jnp.float32,
                        precision=lax.Precision.HIGHEST)
    a_ref[0] = a * _INV_SQRT_C


def _affinity(q, kt):
    bm, bn = 512, 2048
    grid = (_B, _N // bm, _N // bn)
    return pl.pallas_call(
        _aff_body,
        grid=grid,
        in_specs=[
            pl.BlockSpec((1, bm, _C), lambda b, i, j: (b, i, 0)),
            pl.BlockSpec((1, _C, bn), lambda b, i, j: (b, 0, j)),
        ],
        out_specs=pl.BlockSpec((1, bm, bn), lambda b, i, j: (b, i, j)),
        out_shape=jax.ShapeDtypeStruct((_B, _N, _N), jnp.float32),
    )(q, kt)


def _idx_body(loc_ref, rand_ref, out_ref):
    rb = out_ref.shape[0]
    row = jax.lax.broadcasted_iota(jnp.int32, (rb, _NRAND), 0)
    out_ref[:, 0:_NLOC] = loc_ref[...]
    out_ref[:, _NLOC:_S] = ((row & (_RCHUNK - 1)) << 12) + rand_ref[0]


def _assemble_inds(local_tab, rand):
    rb = 2048
    grid = (_B, _N // rb)
    return pl.pallas_call(
        _idx_body,
        grid=grid,
        in_specs=[
            pl.BlockSpec((rb, _NLOC), lambda b, i: (i, 0)),
            pl.BlockSpec((1, rb, _NRAND), lambda b, i: (b, i, 0)),
        ],
        out_specs=pl.BlockSpec((rb, _S), lambda b, i: (b * (_N // rb) + i, 0)),
        out_shape=jax.ShapeDtypeStruct((_ROWS, _S), jnp.int32),
    )(local_tab, rand)


def _sc_body(a_hbm, idx_hbm, segtab_hbm, outl_hbm, outs_hbm,
             a_v0, a_v1, idx_v0, idx_v1, segtab_v,
             outl_v0, outl_v1, outs_v0, outs_v1,
             sin0, sin1, sout0, sout1):
    wid = lax.axis_index("s") * 2 + lax.axis_index("c")
    batch = wid // 16
    pltpu.sync_copy(segtab_hbm.at[batch], segtab_v)
    row0 = wid * _ROWS_PER_W

    bufs = ((a_v0, idx_v0, outl_v0, outs_v0, sin0, sout0),
            (a_v1, idx_v1, outl_v1, outs_v1, sin1, sout1))

    def start_in(ci, buf):
        a_v, idx_v, _, _, sin, _ = buf
        base = row0 + ci * _RCHUNK
        pltpu.async_copy(a_hbm.at[pl.ds(base, _RCHUNK)], a_v, sin)
        pltpu.async_copy(idx_hbm.at[pl.ds(base, _RCHUNK)], idx_v, sin)

    def wait_in(ci, buf):
        a_v, idx_v, _, _, sin, _ = buf
        base = row0 + ci * _RCHUNK
        pltpu.make_async_copy(a_hbm.at[pl.ds(base, _RCHUNK)], a_v, sin).wait()
        pltpu.make_async_copy(idx_hbm.at[pl.ds(base, _RCHUNK)], idx_v,
                              sin).wait()

    def start_out(ci, buf):
        _, _, outl_v, outs_v, _, sout = buf
        base = row0 + ci * _RCHUNK
        pltpu.async_copy(outl_v, outl_hbm.at[pl.ds(base, _RCHUNK)], sout)
        pltpu.async_copy(outs_v, outs_hbm.at[pl.ds(base, _RCHUNK)], sout)

    def wait_out(ci, buf):
        _, _, outl_v, outs_v, _, sout = buf
        base = row0 + ci * _RCHUNK
        pltpu.make_async_copy(outl_v, outl_hbm.at[pl.ds(base, _RCHUNK)],
                              sout).wait()
        pltpu.make_async_copy(outs_v, outs_hbm.at[pl.ds(base, _RCHUNK)],
                              sout).wait()

    def compute(buf):
        a_v, idx_v, outl_v, outs_v, _, _ = buf

        def row_body(r, _):
            def grp_body(g, _):
                sl = pl.ds(g * 16, 16)
                addrs = idx_v[r, sl]
                hi = addrs >> 12
                lo = addrs & 4095
                outl_v[r, sl] = plsc.load_gather(a_v, [hi, lo])
                outs_v[r, sl] = plsc.load_gather(segtab_v, [hi, lo])
                return 0

            lax.fori_loop(0, _NGROUP, grp_body, 0, unroll=8)
            return 0

        lax.fori_loop(0, _RCHUNK, row_body, 0)

    start_in(0, bufs[0])

    def pair_body(i, _):
        c0 = i * 2
        start_in(c0 + 1, bufs[1])
        wait_in(c0, bufs[0])

        @pl.when(i > 0)
        def _():
            wait_out(c0 - 2, bufs[0])

        compute(bufs[0])
        start_out(c0, bufs[0])

        @pl.when(i < _NCHUNK // 2 - 1)
        def _():
            start_in(c0 + 2, bufs[0])

        wait_in(c0 + 1, bufs[1])

        @pl.when(i > 0)
        def _():
            wait_out(c0 - 1, bufs[1])

        compute(bufs[1])
        start_out(c0 + 1, bufs[1])
        return 0

    lax.fori_loop(0, _NCHUNK // 2, pair_body, 0)
    wait_out(_NCHUNK - 2, bufs[0])
    wait_out(_NCHUNK - 1, bufs[1])


def _sc_gather(a2, sample_addrs, segtab):
    fn = functools.partial(
        pl.kernel,
        mesh=plsc.VectorSubcoreMesh(core_axis_name="c", subcore_axis_name="s"),
        compiler_params=pltpu.CompilerParams(needs_layout_passes=False),
        out_type=[
            jax.ShapeDtypeStruct((_ROWS, _S), jnp.float32),
            jax.ShapeDtypeStruct((_ROWS, _S), jnp.int32),
        ],
        scratch_types=[
            pltpu.VMEM((_RCHUNK, _N), jnp.float32),
            pltpu.VMEM((_RCHUNK, _N), jnp.float32),
            pltpu.VMEM((_RCHUNK, _S), jnp.int32),
            pltpu.VMEM((_RCHUNK, _S), jnp.int32),
            pltpu.VMEM((_SEGROWS, _N), jnp.int32),
            pltpu.VMEM((_RCHUNK, _S), jnp.float32),
            pltpu.VMEM((_RCHUNK, _S), jnp.float32),
            pltpu.VMEM((_RCHUNK, _S), jnp.int32),
            pltpu.VMEM((_RCHUNK, _S), jnp.int32),
            pltpu.SemaphoreType.DMA,
            pltpu.SemaphoreType.DMA,
            pltpu.SemaphoreType.DMA,
            pltpu.SemaphoreType.DMA,
        ],
    )(_sc_body)
    return fn(a2, sample_addrs, segtab)


def _loss_body(l_ref, s_ref, seg_ref, out_ref, acc_ref):
    step = pl.program_id(0)
    nsteps = pl.num_programs(0)

    @pl.when(step == 0)
    def _():
        acc_ref[0] = 0.0
        acc_ref[1] = 0.0

    logits = l_ref[...]
    samples = s_ref[...]
    seg = seg_ref[...]
    mask = 1.0 - (seg == 0).astype(jnp.float32)
    targets = (samples == seg).astype(jnp.float32)
    maxes = jnp.max(logits, axis=-1, keepdims=True)
    x_exp = jnp.exp(logits - maxes)
    x_exp_w = x_exp * mask
    denom = jnp.sum(x_exp_w, axis=-1, keepdims=True) + 1e-9
    y_pred = x_exp_w / denom
    y_pred_log = jnp.log(jnp.clip(y_pred, 1e-8, None))
    t_sum = jnp.sum(targets, axis=-1, keepdims=True) + 1e-9
    y_true = targets / t_sum
    kl = jnp.where(y_true > 0,
                   y_true * (jnp.log(jnp.clip(y_true, 1e-30, None)) - y_pred_log),
                   0.0) * mask
    acc_ref[0] += jnp.sum(kl)
    acc_ref[1] += jnp.sum(mask)

    @pl.when(step == nsteps - 1)
    def _():
        out_ref[0, 0] = acc_ref[0] / (acc_ref[1] + 1e-9)


def _loss(logits_flat, samples, seg_col):
    rb = 512
    grid = (_ROWS // rb,)
    return pl.pallas_call(
        _loss_body,
        grid=grid,
        in_specs=[
            pl.BlockSpec((rb, _S), lambda i: (i, 0)),
            pl.BlockSpec((rb, _S), lambda i: (i, 0)),
            pl.BlockSpec((rb, 1), lambda i: (i, 0)),
        ],
        out_specs=pl.BlockSpec((1, 1), lambda i: (0, 0),
                               memory_space=pltpu.SMEM),
        out_shape=jax.ShapeDtypeStruct((1, 1), jnp.float32),
        scratch_shapes=[pltpu.SMEM((2,), jnp.float32)],
    )(logits_flat, samples, seg_col)


def kernel(features, segment_target, rand_global_inds,
           W_feat, b_feat, W_key, b_key, W_query, b_query):
    x = features.reshape(_B, _IN_DIM, _N)
    wqc = jnp.dot(W_query, W_feat, precision=lax.Precision.HIGHEST)
    wkc = jnp.dot(W_key, W_feat, precision=lax.Precision.HIGHEST)
    qbias = (jnp.dot(W_query, b_feat) + b_query).reshape(1, _C)
    kbias = (jnp.dot(W_key, b_feat) + b_key).reshape(_C, 1)

    q, kt = _projections(x, wqc, wkc, qbias, kbias)
    a = _affinity(q, kt).reshape(_ROWS, _N)

    sample_addrs = _assemble_inds(jnp.asarray(_LOCAL_ADDR),
                                  rand_global_inds.astype(jnp.int32))
    seg_flat = segment_target.reshape(_ROWS).astype(jnp.int32)
    segtab = jnp.broadcast_to(seg_flat.reshape(_B, 1, _N), (_B, _SEGROWS, _N))

    logits_flat, samples = _sc_gather(a, sample_addrs, segtab)

    loss2d = _loss(logits_flat, samples, seg_flat.reshape(_ROWS, 1))
    loss = loss2d[0, 0]
    return (logits_flat.reshape(_B, _N, _S), loss)


# speedup vs baseline: 375.0009x; 1.0280x over previous
import functools

import numpy as np
import jax
import jax.numpy as jnp
from jax import lax
from jax.experimental import pallas as pl
from jax.experimental.pallas import tpu as pltpu
from jax.experimental.pallas import tpu_sc as plsc

_B = 2
_IN_DIM = 256
_C = 32
_H = _W = 64
_N = _H * _W
_KWIN = 25
_S = 1024
_NLOC = _KWIN * _KWIN
_NRAND = _S - _NLOC
_ROWS = _B * _N
_INV_SQRT_C = float(_C) ** -0.5

_NUM_WORKERS = 32
_ROWS_PER_W = _ROWS // _NUM_WORKERS
_RCHUNK = 4
_NCHUNK = _ROWS_PER_W // _RCHUNK
_NGROUP = _S // 16
_SEGROWS = _RCHUNK


def _local_window_inds():
    half = (_KWIN - 1) // 2
    idx = np.arange(_N, dtype=np.int64).reshape(_H, _W)
    padded = np.zeros((_H + 2 * half, _W + 2 * half), dtype=np.int64)
    padded[half:half + _H, half:half + _W] = idx
    win = np.lib.stride_tricks.sliding_window_view(padded, (_KWIN, _KWIN))
    return win.reshape(_N, _NLOC).astype(np.int32)


_LOCAL_INDS = _local_window_inds()
_LOCAL_ADDR = (((np.arange(_N, dtype=np.int32)[:, None] % _RCHUNK) << 12)
               + _LOCAL_INDS)


def _proj_body(x_ref, wq_ref, wk_ref, qb_ref, kb_ref, q_ref, kt_ref):
    x = x_ref[0]
    q = lax.dot_general(x, wq_ref[...], (((0,), (1,)), ((), ())),
                        preferred_element_type=jnp.float32,
                        precision=lax.Precision.HIGHEST)
    q_ref[0] = q + qb_ref[...]
    kt = lax.dot_general(wk_ref[...], x, (((1,), (0,)), ((), ())),
                         preferred_element_type=jnp.float32,
                         precision=lax.Precision.HIGHEST)
    kt_ref[0] = kt + kb_ref[...]


def _projections(x, wqc, wkc, qbias, kbias):
    bn = 2048
    grid = (_B, _N // bn)
    return pl.pallas_call(
        _proj_body,
        grid=grid,
        in_specs=[
            pl.BlockSpec((1, _IN_DIM, bn), lambda b, j: (b, 0, j)),
            pl.BlockSpec((_C, _IN_DIM), lambda b, j: (0, 0)),
            pl.BlockSpec((_C, _IN_DIM), lambda b, j: (0, 0)),
            pl.BlockSpec((1, _C), lambda b, j: (0, 0)),
            pl.BlockSpec((_C, 1), lambda b, j: (0, 0)),
        ],
        out_specs=[
            pl.BlockSpec((1, bn, _C), lambda b, j: (b, j, 0)),
            pl.BlockSpec((1, _C, bn), lambda b, j: (b, 0, j)),
        ],
        out_shape=[
            jax.ShapeDtypeStruct((_B, _N, _C), jnp.float32),
            jax.ShapeDtypeStruct((_B, _C, _N), jnp.float32),
        ],
    )(x, wqc, wkc, qbias, kbias)


def _aff_body(q_ref, kt_ref, a_ref):
    a = lax.dot_general(q_ref[0], kt_ref[0], (((1,), (0,)), ((), ())),
                        preferred_element_type=jnp.float32,
                        precision=lax.Precision.HIGHEST)
    a_ref[0] = a * _INV_SQRT_C


def _affinity(q, kt):
    bm, bn = 512, 2048
    grid = (_B, _N // bm, _N // bn)
    return pl.pallas_call(
        _aff_body,
        grid=grid,
        in_specs=[
            pl.BlockSpec((1, bm, _C), lambda b, i, j: (b, i, 0)),
            pl.BlockSpec((1, _C, bn), lambda b, i, j: (b, 0, j)),
        ],
        out_specs=pl.BlockSpec((1, bm, bn), lambda b, i, j: (b, i, j)),
        out_shape=jax.ShapeDtypeStruct((_B, _N, _N), jnp.float32),
    )(q, kt)


def _idx_body(loc_ref, rand_ref, out_ref):
    rb = out_ref.shape[0]
    row = jax.lax.broadcasted_iota(jnp.int32, (rb, _NRAND), 0)
    out_ref[:, 0:_NLOC] = loc_ref[...]
    out_ref[:, _NLOC:_S] = ((row & (_RCHUNK - 1)) << 12) + rand_ref[0]


def _assemble_inds(local_tab, rand):
    rb = 2048
    grid = (_B, _N // rb)
    return pl.pallas_call(
        _idx_body,
        grid=grid,
        in_specs=[
            pl.BlockSpec((rb, _NLOC), lambda b, i: (i, 0)),
            pl.BlockSpec((1, rb, _NRAND), lambda b, i: (b, i, 0)),
        ],
        out_specs=pl.BlockSpec((rb, _S), lambda b, i: (b * (_N // rb) + i, 0)),
        out_shape=jax.ShapeDtypeStruct((_ROWS, _S), jnp.int32),
    )(local_tab, rand)


def _sc_body(a_hbm, idx_hbm, segtab_hbm, outl_hbm, outs_hbm,
             a_v0, a_v1, idx_v0, idx_v1, segtab_v,
             outl_v0, outl_v1, outs_v0, outs_v1,
             sin0, sin1, sout0, sout1):
    wid = lax.axis_index("s") * 2 + lax.axis_index("c")
    batch = wid // 16
    pltpu.sync_copy(segtab_hbm.at[batch], segtab_v)
    row0 = wid * _ROWS_PER_W

    bufs = ((a_v0, idx_v0, outl_v0, outs_v0, sin0, sout0),
            (a_v1, idx_v1, outl_v1, outs_v1, sin1, sout1))

    def start_in(ci, buf):
        a_v, idx_v, _, _, sin, _ = buf
        base = row0 + ci * _RCHUNK
        pltpu.async_copy(a_hbm.at[pl.ds(base, _RCHUNK)], a_v, sin)
        pltpu.async_copy(idx_hbm.at[pl.ds(base, _RCHUNK)], idx_v, sin)

    def wait_in(ci, buf):
        a_v, idx_v, _, _, sin, _ = buf
        base = row0 + ci * _RCHUNK
        pltpu.make_async_copy(a_hbm.at[pl.ds(base, _RCHUNK)], a_v, sin).wait()
        pltpu.make_async_copy(idx_hbm.at[pl.ds(base, _RCHUNK)], idx_v,
                              sin).wait()

    def start_out(ci, buf):
        _, _, outl_v, outs_v, _, sout = buf
        base = row0 + ci * _RCHUNK
        pltpu.async_copy(outl_v, outl_hbm.at[pl.ds(base, _RCHUNK)], sout)
        pltpu.async_copy(outs_v, outs_hbm.at[pl.ds(base, _RCHUNK)], sout)

    def wait_out(ci, buf):
        _, _, outl_v, outs_v, _, sout = buf
        base = row0 + ci * _RCHUNK
        pltpu.make_async_copy(outl_v, outl_hbm.at[pl.ds(base, _RCHUNK)],
                              sout).wait()
        pltpu.make_async_copy(outs_v, outs_hbm.at[pl.ds(base, _RCHUNK)],
                              sout).wait()

    def compute(buf):
        a_v, idx_v, outl_v, outs_v, _, _ = buf

        def row_body(r, _):
            def grp_body(g, _):
                sl = pl.ds(g * 16, 16)
                addrs = idx_v[r, sl]
                hi = addrs >> 12
                lo = addrs & 4095
                outl_v[r, sl] = plsc.load_gather(a_v, [hi, lo])
                outs_v[r, sl] = plsc.load_gather(segtab_v, [hi, lo])
                return 0

            lax.fori_loop(0, _NGROUP, grp_body, 0, unroll=8)
            return 0

        lax.fori_loop(0, _RCHUNK, row_body, 0)

    start_in(0, bufs[0])

    def pair_body(i, _):
        c0 = i * 2
        start_in(c0 + 1, bufs[1])
        wait_in(c0, bufs[0])

        @pl.when(i > 0)
        def _():
            wait_out(c0 - 2, bufs[0])

        compute(bufs[0])
        start_out(c0, bufs[0])

        @pl.when(i < _NCHUNK // 2 - 1)
        def _():
            start_in(c0 + 2, bufs[0])

        wait_in(c0 + 1, bufs[1])

        @pl.when(i > 0)
        def _():
            wait_out(c0 - 1, bufs[1])

        compute(bufs[1])
        start_out(c0 + 1, bufs[1])
        return 0

    lax.fori_loop(0, _NCHUNK // 2, pair_body, 0)
    wait_out(_NCHUNK - 2, bufs[0])
    wait_out(_NCHUNK - 1, bufs[1])


def _sc_gather(a2, sample_addrs, segtab):
    fn = functools.partial(
        pl.kernel,
        mesh=plsc.VectorSubcoreMesh(core_axis_name="c", subcore_axis_name="s"),
        compiler_params=pltpu.CompilerParams(needs_layout_passes=False),
        out_type=[
            jax.ShapeDtypeStruct((_ROWS, _S), jnp.float32),
            jax.ShapeDtypeStruct((_ROWS, _S), jnp.int32),
        ],
        scratch_types=[
            pltpu.VMEM((_RCHUNK, _N), jnp.float32),
            pltpu.VMEM((_RCHUNK, _N), jnp.float32),
            pltpu.VMEM((_RCHUNK, _S), jnp.int32),
            pltpu.VMEM((_RCHUNK, _S), jnp.int32),
            pltpu.VMEM((_SEGROWS, _N), jnp.int32),
            pltpu.VMEM((_RCHUNK, _S), jnp.float32),
            pltpu.VMEM((_RCHUNK, _S), jnp.float32),
            pltpu.VMEM((_RCHUNK, _S), jnp.int32),
            pltpu.VMEM((_RCHUNK, _S), jnp.int32),
            pltpu.SemaphoreType.DMA,
            pltpu.SemaphoreType.DMA,
            pltpu.SemaphoreType.DMA,
            pltpu.SemaphoreType.DMA,
        ],
    )(_sc_body)
    return fn(a2, sample_addrs, segtab)


def _loss_body(l_ref, s_ref, seg_ref, out_ref, acc_ref):
    step = pl.program_id(0)
    nsteps = pl.num_programs(0)

    @pl.when(step == 0)
    def _():
        acc_ref[0] = 0.0
        acc_ref[1] = 0.0

    logits = l_ref[...]
    samples = s_ref[...]
    seg = seg_ref[...]
    mask = 1.0 - (seg == 0).astype(jnp.float32)
    targets = (samples == seg).astype(jnp.float32)
    maxes = jnp.max(logits, axis=-1, keepdims=True)
    shifted = logits - maxes
    denom = jnp.sum(jnp.exp(shifted), axis=-1, keepdims=True) * mask + 1e-9
    y_pred_log = jnp.maximum(shifted - jnp.log(denom),
                             jnp.float32(np.log(1e-8)))
    t_sum = jnp.sum(targets, axis=-1, keepdims=True) + 1e-9
    tl_sum = jnp.sum(targets * y_pred_log, axis=-1, keepdims=True)
    t_cnt = jnp.sum(targets, axis=-1, keepdims=True)
    kl_row = (mask / t_sum) * (-jnp.log(t_sum) * t_cnt - tl_sum)
    acc_ref[0] += jnp.sum(kl_row)
    acc_ref[1] += jnp.sum(mask)

    @pl.when(step == nsteps - 1)
    def _():
        out_ref[0, 0] = acc_ref[0] / (acc_ref[1] + 1e-9)


def _loss(logits_flat, samples, seg_col):
    rb = 512
    grid = (_ROWS // rb,)
    return pl.pallas_call(
        _loss_body,
        grid=grid,
        in_specs=[
            pl.BlockSpec((rb, _S), lambda i: (i, 0)),
            pl.BlockSpec((rb, _S), lambda i: (i, 0)),
            pl.BlockSpec((rb, 1), lambda i: (i, 0)),
        ],
        out_specs=pl.BlockSpec((1, 1), lambda i: (0, 0),
                               memory_space=pltpu.SMEM),
        out_shape=jax.ShapeDtypeStruct((1, 1), jnp.float32),
        scratch_shapes=[pltpu.SMEM((2,), jnp.float32)],
    )(logits_flat, samples, seg_col)


def kernel(features, segment_target, rand_global_inds,
           W_feat, b_feat, W_key, b_key, W_query, b_query):
    x = features.reshape(_B, _IN_DIM, _N)
    wqc = jnp.dot(W_query, W_feat, precision=lax.Precision.HIGHEST)
    wkc = jnp.dot(W_key, W_feat, precision=lax.Precision.HIGHEST)
    qbias = (jnp.dot(W_query, b_feat) + b_query).reshape(1, _C)
    kbias = (jnp.dot(W_key, b_feat) + b_key).reshape(_C, 1)

    q, kt = _projections(x, wqc, wkc, qbias, kbias)
    a = _affinity(q, kt).reshape(_ROWS, _N)

    sample_addrs = _assemble_inds(jnp.asarray(_LOCAL_ADDR),
                                  rand_global_inds.astype(jnp.int32))
    seg_flat = segment_target.reshape(_ROWS).astype(jnp.int32)
    segtab = jnp.broadcast_to(seg_flat.reshape(_B, 1, _N), (_B, _SEGROWS, _N))

    logits_flat, samples = _sc_gather(a, sample_addrs, segtab)

    loss2d = _loss(logits_flat, samples, seg_flat.reshape(_ROWS, 1))
    loss = loss2d[0, 0]
    return (logits_flat.reshape(_B, _N, _S), loss)


# speedup vs baseline: 389.1227x; 1.0377x over previous
import functools

import numpy as np
import jax
import jax.numpy as jnp
from jax import lax
from jax.experimental import pallas as pl
from jax.experimental.pallas import tpu as pltpu
from jax.experimental.pallas import tpu_sc as plsc

_B = 2
_IN_DIM = 256
_C = 32
_H = _W = 64
_N = _H * _W
_KWIN = 25
_S = 1024
_NLOC = _KWIN * _KWIN
_NRAND = _S - _NLOC
_ROWS = _B * _N
_INV_SQRT_C = float(_C) ** -0.5

_NUM_WORKERS = 32
_ROWS_PER_W = _N // _NUM_WORKERS
_RCHUNK = 4
_NCHUNK = _ROWS_PER_W // _RCHUNK
_NGROUP = _S // 16
_SEGROWS = _RCHUNK


def _local_window_inds():
    half = (_KWIN - 1) // 2
    idx = np.arange(_N, dtype=np.int64).reshape(_H, _W)
    padded = np.zeros((_H + 2 * half, _W + 2 * half), dtype=np.int64)
    padded[half:half + _H, half:half + _W] = idx
    win = np.lib.stride_tricks.sliding_window_view(padded, (_KWIN, _KWIN))
    return win.reshape(_N, _NLOC).astype(np.int32)


_LOCAL_INDS = _local_window_inds()
_LOCAL_ADDR = (((np.arange(_N, dtype=np.int32)[:, None] % _RCHUNK) << 12)
               + _LOCAL_INDS)


def _proj_body(x_ref, wq_ref, wk_ref, qb_ref, kb_ref, q_ref, kt_ref):
    x = x_ref[0]
    q = lax.dot_general(x, wq_ref[...], (((0,), (1,)), ((), ())),
                        preferred_element_type=jnp.float32,
                        precision=lax.Precision.HIGHEST)
    q_ref[0] = q + qb_ref[...]
    kt = lax.dot_general(wk_ref[...], x, (((1,), (0,)), ((), ())),
                         preferred_element_type=jnp.float32,
                         precision=lax.Precision.HIGHEST)
    kt_ref[0] = kt + kb_ref[...]


def _projections(x, wqc, wkc, qbias, kbias):
    bn = 2048
    grid = (_B, _N // bn)
    return pl.pallas_call(
        _proj_body,
        grid=grid,
        in_specs=[
            pl.BlockSpec((1, _IN_DIM, bn), lambda b, j: (b, 0, j)),
            pl.BlockSpec((_C, _IN_DIM), lambda b, j: (0, 0)),
            pl.BlockSpec((_C, _IN_DIM), lambda b, j: (0, 0)),
            pl.BlockSpec((1, _C), lambda b, j: (0, 0)),
            pl.BlockSpec((_C, 1), lambda b, j: (0, 0)),
        ],
        out_specs=[
            pl.BlockSpec((1, bn, _C), lambda b, j: (b, j, 0)),
            pl.BlockSpec((1, _C, bn), lambda b, j: (b, 0, j)),
        ],
        out_shape=[
            jax.ShapeDtypeStruct((_B, _N, _C), jnp.float32),
            jax.ShapeDtypeStruct((_B, _C, _N), jnp.float32),
        ],
    )(x, wqc, wkc, qbias, kbias)


def _aff_body(q_ref, kt_ref, a_ref):
    a = lax.dot_general(q_ref[0], kt_ref[0], (((1,), (0,)), ((), ())),
                        preferred_element_type=jnp.float32,
                        precision=lax.Precision.HIGHEST)
    a_ref[0] = a * _INV_SQRT_C


def _affinity(qb, ktb):
    bm, bn = 512, 2048
    grid = (_N // bm, _N // bn)
    return pl.pallas_call(
        _aff_body,
        grid=grid,
        in_specs=[
            pl.BlockSpec((1, bm, _C), lambda i, j: (0, i, 0)),
            pl.BlockSpec((1, _C, bn), lambda i, j: (0, 0, j)),
        ],
        out_specs=pl.BlockSpec((1, bm, bn), lambda i, j: (0, i, j)),
        out_shape=jax.ShapeDtypeStruct((1, _N, _N), jnp.float32),
    )(qb, ktb)


def _idx_body(loc_ref, rand_ref, out_ref):
    rb = out_ref.shape[0]
    row = jax.lax.broadcasted_iota(jnp.int32, (rb, _NRAND), 0)
    out_ref[:, 0:_NLOC] = loc_ref[...]
    out_ref[:, _NLOC:_S] = ((row & (_RCHUNK - 1)) << 12) + rand_ref[0]


def _assemble_inds(local_tab, rand_b):
    rb = 2048
    grid = (_N // rb,)
    return pl.pallas_call(
        _idx_body,
        grid=grid,
        in_specs=[
            pl.BlockSpec((rb, _NLOC), lambda i: (i, 0)),
            pl.BlockSpec((1, rb, _NRAND), lambda i: (0, i, 0)),
        ],
        out_specs=pl.BlockSpec((rb, _S), lambda i: (i, 0)),
        out_shape=jax.ShapeDtypeStruct((_N, _S), jnp.int32),
    )(local_tab, rand_b)


def _sc_body(a_hbm, idx_hbm, segtab_hbm, outl_hbm, outs_hbm,
             a_v0, a_v1, idx_v0, idx_v1, segtab_v,
             outl_v0, outl_v1, outs_v0, outs_v1,
             sin0, sin1, sout0, sout1):
    wid = lax.axis_index("s") * 2 + lax.axis_index("c")
    pltpu.sync_copy(segtab_hbm, segtab_v)
    row0 = wid * _ROWS_PER_W

    bufs = ((a_v0, idx_v0, outl_v0, outs_v0, sin0, sout0),
            (a_v1, idx_v1, outl_v1, outs_v1, sin1, sout1))

    def start_in(ci, buf):
        a_v, idx_v, _, _, sin, _ = buf
        base = row0 + ci * _RCHUNK
        pltpu.async_copy(a_hbm.at[pl.ds(base, _RCHUNK)], a_v, sin)
        pltpu.async_copy(idx_hbm.at[pl.ds(base, _RCHUNK)], idx_v, sin)

    def wait_in(ci, buf):
        a_v, idx_v, _, _, sin, _ = buf
        base = row0 + ci * _RCHUNK
        pltpu.make_async_copy(a_hbm.at[pl.ds(base, _RCHUNK)], a_v, sin).wait()
        pltpu.make_async_copy(idx_hbm.at[pl.ds(base, _RCHUNK)], idx_v,
                              sin).wait()

    def start_out(ci, buf):
        _, _, outl_v, outs_v, _, sout = buf
        base = row0 + ci * _RCHUNK
        pltpu.async_copy(outl_v, outl_hbm.at[pl.ds(base, _RCHUNK)], sout)
        pltpu.async_copy(outs_v, outs_hbm.at[pl.ds(base, _RCHUNK)], sout)

    def wait_out(ci, buf):
        _, _, outl_v, outs_v, _, sout = buf
        base = row0 + ci * _RCHUNK
        pltpu.make_async_copy(outl_v, outl_hbm.at[pl.ds(base, _RCHUNK)],
                              sout).wait()
        pltpu.make_async_copy(outs_v, outs_hbm.at[pl.ds(base, _RCHUNK)],
                              sout).wait()

    def compute(buf):
        a_v, idx_v, outl_v, outs_v, _, _ = buf

        def row_body(r, _):
            def grp_body(g, _):
                sl = pl.ds(g * 16, 16)
                addrs = idx_v[r, sl]
                hi = addrs >> 12
                lo = addrs & 4095
                outl_v[r, sl] = plsc.load_gather(a_v, [hi, lo])
                outs_v[r, sl] = plsc.load_gather(segtab_v, [hi, lo])
                return 0

            lax.fori_loop(0, _NGROUP, grp_body, 0, unroll=8)
            return 0

        lax.fori_loop(0, _RCHUNK, row_body, 0)

    start_in(0, bufs[0])

    def pair_body(i, _):
        c0 = i * 2
        start_in(c0 + 1, bufs[1])
        wait_in(c0, bufs[0])

        @pl.when(i > 0)
        def _():
            wait_out(c0 - 2, bufs[0])

        compute(bufs[0])
        start_out(c0, bufs[0])

        @pl.when(i < _NCHUNK // 2 - 1)
        def _():
            start_in(c0 + 2, bufs[0])

        wait_in(c0 + 1, bufs[1])

        @pl.when(i > 0)
        def _():
            wait_out(c0 - 1, bufs[1])

        compute(bufs[1])
        start_out(c0 + 1, bufs[1])
        return 0

    lax.fori_loop(0, _NCHUNK // 2, pair_body, 0)
    wait_out(_NCHUNK - 2, bufs[0])
    wait_out(_NCHUNK - 1, bufs[1])


def _sc_gather(a2, sample_addrs, segtab):
    fn = functools.partial(
        pl.kernel,
        mesh=plsc.VectorSubcoreMesh(core_axis_name="c", subcore_axis_name="s"),
        compiler_params=pltpu.CompilerParams(needs_layout_passes=False),
        out_type=[
            jax.ShapeDtypeStruct((_N, _S), jnp.float32),
            jax.ShapeDtypeStruct((_N, _S), jnp.int32),
        ],
        scratch_types=[
            pltpu.VMEM((_RCHUNK, _N), jnp.float32),
            pltpu.VMEM((_RCHUNK, _N), jnp.float32),
            pltpu.VMEM((_RCHUNK, _S), jnp.int32),
            pltpu.VMEM((_RCHUNK, _S), jnp.int32),
            pltpu.VMEM((_SEGROWS, _N), jnp.int32),
            pltpu.VMEM((_RCHUNK, _S), jnp.float32),
            pltpu.VMEM((_RCHUNK, _S), jnp.float32),
            pltpu.VMEM((_RCHUNK, _S), jnp.int32),
            pltpu.VMEM((_RCHUNK, _S), jnp.int32),
            pltpu.SemaphoreType.DMA,
            pltpu.SemaphoreType.DMA,
            pltpu.SemaphoreType.DMA,
            pltpu.SemaphoreType.DMA,
        ],
    )(_sc_body)
    return fn(a2, sample_addrs, segtab)


def _loss_body(l_ref, s_ref, seg_ref, out_ref, acc_ref):
    step = pl.program_id(0)
    nsteps = pl.num_programs(0)

    @pl.when(step == 0)
    def _():
        acc_ref[0] = 0.0
        acc_ref[1] = 0.0

    logits = l_ref[...]
    samples = s_ref[...]
    seg = seg_ref[...]
    mask = 1.0 - (seg == 0).astype(jnp.float32)
    targets = (samples == seg).astype(jnp.float32)
    maxes = jnp.max(logits, axis=-1, keepdims=True)
    shifted = logits - maxes
    denom = jnp.sum(jnp.exp(shifted), axis=-1, keepdims=True) * mask + 1e-9
    y_pred_log = jnp.maximum(shifted - jnp.log(denom),
                             jnp.float32(np.log(1e-8)))
    t_sum = jnp.sum(targets, axis=-1, keepdims=True) + 1e-9
    tl_sum = jnp.sum(targets * y_pred_log, axis=-1, keepdims=True)
    t_cnt = jnp.sum(targets, axis=-1, keepdims=True)
    kl_row = (mask / t_sum) * (-jnp.log(t_sum) * t_cnt - tl_sum)
    acc_ref[0] += jnp.sum(kl_row)
    acc_ref[1] += jnp.sum(mask)

    @pl.when(step == nsteps - 1)
    def _():
        out_ref[0, 0] = acc_ref[0]
        out_ref[0, 1] = acc_ref[1]


def _loss_partial(logits_b, samples_b, seg_col_b):
    rb = 512
    grid = (_N // rb,)
    return pl.pallas_call(
        _loss_body,
        grid=grid,
        in_specs=[
            pl.BlockSpec((rb, _S), lambda i: (i, 0)),
            pl.BlockSpec((rb, _S), lambda i: (i, 0)),
            pl.BlockSpec((rb, 1), lambda i: (i, 0)),
        ],
        out_specs=pl.BlockSpec((1, 2), lambda i: (0, 0),
                               memory_space=pltpu.SMEM),
        out_shape=jax.ShapeDtypeStruct((1, 2), jnp.float32),
        scratch_shapes=[pltpu.SMEM((2,), jnp.float32)],
    )(logits_b, samples_b, seg_col_b)


def kernel(features, segment_target, rand_global_inds,
           W_feat, b_feat, W_key, b_key, W_query, b_query):
    x = features.reshape(_B, _IN_DIM, _N)
    wqc = jnp.dot(W_query, W_feat, precision=lax.Precision.HIGHEST)
    wkc = jnp.dot(W_key, W_feat, precision=lax.Precision.HIGHEST)
    qbias = (jnp.dot(W_query, b_feat) + b_query).reshape(1, _C)
    kbias = (jnp.dot(W_key, b_feat) + b_key).reshape(_C, 1)

    q, kt = _projections(x, wqc, wkc, qbias, kbias)
    rand32 = rand_global_inds.astype(jnp.int32)
    seg = segment_target.reshape(_B, _N).astype(jnp.int32)
    local_tab = jnp.asarray(_LOCAL_ADDR)

    logits_halves, partials = [], []
    for b in range(_B):
        a_b = _affinity(q[b:b + 1], kt[b:b + 1]).reshape(_N, _N)
        idx_b = _assemble_inds(local_tab, rand32[b:b + 1])
        segtab_b = jnp.broadcast_to(seg[b].reshape(1, _N), (_SEGROWS, _N))
        logits_b, samples_b = _sc_gather(a_b, idx_b, segtab_b)
        logits_halves.append(logits_b)
        partials.append(_loss_partial(logits_b, samples_b,
                                      seg[b].reshape(_N, 1)))

    psum = partials[0] + partials[1]
    loss = psum[0, 0] / (psum[0, 1] + 1e-9)
    logits = jnp.stack(logits_halves).reshape(_B, _N, _S)
    return (logits, loss)


# speedup vs baseline: 497.1597x; 1.2776x over previous
import functools

import numpy as np
import jax
import jax.numpy as jnp
from jax import lax
from jax.experimental import pallas as pl
from jax.experimental.pallas import tpu as pltpu
from jax.experimental.pallas import tpu_sc as plsc

_B = 2
_IN_DIM = 256
_C = 32
_H = _W = 64
_N = _H * _W
_KWIN = 25
_S = 1024
_NLOC = _KWIN * _KWIN
_NRAND = _S - _NLOC
_ROWS = _B * _N
_INV_SQRT_C = float(_C) ** -0.5

_NUM_WORKERS = 32
_ROWS_PER_W = _N // _NUM_WORKERS
_RCHUNK = 4
_NCHUNK = _ROWS_PER_W // _RCHUNK
_NGROUP = _S // 16
_SEGROWS = _RCHUNK


def _local_window_inds():
    half = (_KWIN - 1) // 2
    idx = np.arange(_N, dtype=np.int64).reshape(_H, _W)
    padded = np.zeros((_H + 2 * half, _W + 2 * half), dtype=np.int64)
    padded[half:half + _H, half:half + _W] = idx
    win = np.lib.stride_tricks.sliding_window_view(padded, (_KWIN, _KWIN))
    return win.reshape(_N, _NLOC).astype(np.int32)


_LOCAL_INDS = _local_window_inds()
_LOCAL_ADDR = (((np.arange(_N, dtype=np.int32)[:, None] % _RCHUNK) << 12)
               + _LOCAL_INDS)


def _proj_body(x_ref, wq_ref, wk_ref, qb_ref, kb_ref, q_ref, kt_ref):
    x = x_ref[0]
    q = lax.dot_general(x, wq_ref[...], (((0,), (1,)), ((), ())),
                        preferred_element_type=jnp.float32,
                        precision=lax.Precision.HIGHEST)
    q_ref[0] = q + qb_ref[...]
    kt = lax.dot_general(wk_ref[...], x, (((1,), (0,)), ((), ())),
                         preferred_element_type=jnp.float32,
                         precision=lax.Precision.HIGHEST)
    kt_ref[0] = kt + kb_ref[...]


def _projections(x, wqc, wkc, qbias, kbias):
    bn = 2048
    grid = (_B, _N // bn)
    return pl.pallas_call(
        _proj_body,
        grid=grid,
        in_specs=[
            pl.BlockSpec((1, _IN_DIM, bn), lambda b, j: (b, 0, j)),
            pl.BlockSpec((_C, _IN_DIM), lambda b, j: (0, 0)),
            pl.BlockSpec((_C, _IN_DIM), lambda b, j: (0, 0)),
            pl.BlockSpec((1, _C), lambda b, j: (0, 0)),
            pl.BlockSpec((_C, 1), lambda b, j: (0, 0)),
        ],
        out_specs=[
            pl.BlockSpec((1, bn, _C), lambda b, j: (b, j, 0)),
            pl.BlockSpec((1, _C, bn), lambda b, j: (b, 0, j)),
        ],
        out_shape=[
            jax.ShapeDtypeStruct((_B, _N, _C), jnp.float32),
            jax.ShapeDtypeStruct((_B, _C, _N), jnp.float32),
        ],
    )(x, wqc, wkc, qbias, kbias)


def _aff_body(q_ref, kt_ref, a_ref):
    a = lax.dot_general(q_ref[0], kt_ref[0], (((1,), (0,)), ((), ())),
                        preferred_element_type=jnp.float32,
                        precision=lax.Precision.HIGHEST)
    a_ref[0] = a * _INV_SQRT_C


def _affinity(qb, ktb):
    bm, bn = 512, 2048
    grid = (_N // bm, _N // bn)
    return pl.pallas_call(
        _aff_body,
        grid=grid,
        in_specs=[
            pl.BlockSpec((1, bm, _C), lambda i, j: (0, i, 0)),
            pl.BlockSpec((1, _C, bn), lambda i, j: (0, 0, j)),
        ],
        out_specs=pl.BlockSpec((1, bm, bn), lambda i, j: (0, i, j)),
        out_shape=jax.ShapeDtypeStruct((1, _N, _N), jnp.float32),
    )(qb, ktb)


def _idx_body(loc_ref, rand_ref, out_ref):
    rb = out_ref.shape[0]
    row = jax.lax.broadcasted_iota(jnp.int32, (rb, _NRAND), 0)
    out_ref[:, 0:_NLOC] = loc_ref[...]
    out_ref[:, _NLOC:_S] = ((row & (_RCHUNK - 1)) << 12) + rand_ref[0]


def _assemble_inds(local_tab, rand_b):
    rb = 2048
    grid = (_N // rb,)
    return pl.pallas_call(
        _idx_body,
        grid=grid,
        in_specs=[
            pl.BlockSpec((rb, _NLOC), lambda i: (i, 0)),
            pl.BlockSpec((1, rb, _NRAND), lambda i: (0, i, 0)),
        ],
        out_specs=pl.BlockSpec((rb, _S), lambda i: (i, 0)),
        out_shape=jax.ShapeDtypeStruct((_N, _S), jnp.int32),
    )(local_tab, rand_b)


def _sc_body(a_hbm, idx_hbm, segtab_hbm, outl_hbm, outs_hbm,
             a_v0, a_v1, idx_v0, idx_v1, segtab_v,
             outl_v0, outl_v1, outs_v0, outs_v1,
             sin0, sin1, sout0, sout1):
    wid = lax.axis_index("s") * 2 + lax.axis_index("c")
    pltpu.sync_copy(segtab_hbm, segtab_v)
    row0 = wid * _ROWS_PER_W

    bufs = ((a_v0, idx_v0, outl_v0, outs_v0, sin0, sout0),
            (a_v1, idx_v1, outl_v1, outs_v1, sin1, sout1))

    def start_in(ci, buf):
        a_v, idx_v, _, _, sin, _ = buf
        base = row0 + ci * _RCHUNK
        pltpu.async_copy(a_hbm.at[pl.ds(base, _RCHUNK)], a_v, sin)
        pltpu.async_copy(idx_hbm.at[pl.ds(base, _RCHUNK)], idx_v, sin)

    def wait_in(ci, buf):
        a_v, idx_v, _, _, sin, _ = buf
        base = row0 + ci * _RCHUNK
        pltpu.make_async_copy(a_hbm.at[pl.ds(base, _RCHUNK)], a_v, sin).wait()
        pltpu.make_async_copy(idx_hbm.at[pl.ds(base, _RCHUNK)], idx_v,
                              sin).wait()

    def start_out(ci, buf):
        _, _, outl_v, outs_v, _, sout = buf
        base = row0 + ci * _RCHUNK
        pltpu.async_copy(outl_v, outl_hbm.at[pl.ds(base, _RCHUNK)], sout)
        pltpu.async_copy(outs_v, outs_hbm.at[pl.ds(base, _RCHUNK)], sout)

    def wait_out(ci, buf):
        _, _, outl_v, outs_v, _, sout = buf
        base = row0 + ci * _RCHUNK
        pltpu.make_async_copy(outl_v, outl_hbm.at[pl.ds(base, _RCHUNK)],
                              sout).wait()
        pltpu.make_async_copy(outs_v, outs_hbm.at[pl.ds(base, _RCHUNK)],
                              sout).wait()

    def compute(buf):
        a_v, idx_v, outl_v, outs_v, _, _ = buf

        def row_body(r, _):
            def grp_body(g):
                sl = pl.ds(g * 16, 16)
                addrs = idx_v[r, sl]
                hi = addrs >> 12
                lo = addrs & 4095
                outl_v[r, sl] = plsc.load_gather(a_v, [hi, lo])
                outs_v[r, sl] = plsc.load_gather(segtab_v, [hi, lo])

            plsc.parallel_loop(0, _NGROUP, 1, unroll=8)(grp_body)
            return 0

        lax.fori_loop(0, _RCHUNK, row_body, 0)

    start_in(0, bufs[0])

    def pair_body(i, _):
        c0 = i * 2
        start_in(c0 + 1, bufs[1])
        wait_in(c0, bufs[0])

        @pl.when(i > 0)
        def _():
            wait_out(c0 - 2, bufs[0])

        compute(bufs[0])
        start_out(c0, bufs[0])

        @pl.when(i < _NCHUNK // 2 - 1)
        def _():
            start_in(c0 + 2, bufs[0])

        wait_in(c0 + 1, bufs[1])

        @pl.when(i > 0)
        def _():
            wait_out(c0 - 1, bufs[1])

        compute(bufs[1])
        start_out(c0 + 1, bufs[1])
        return 0

    lax.fori_loop(0, _NCHUNK // 2, pair_body, 0)
    wait_out(_NCHUNK - 2, bufs[0])
    wait_out(_NCHUNK - 1, bufs[1])


def _sc_gather(a2, sample_addrs, segtab):
    fn = functools.partial(
        pl.kernel,
        mesh=plsc.VectorSubcoreMesh(core_axis_name="c", subcore_axis_name="s"),
        compiler_params=pltpu.CompilerParams(needs_layout_passes=False),
        out_type=[
            jax.ShapeDtypeStruct((_N, _S), jnp.float32),
            jax.ShapeDtypeStruct((_N, _S), jnp.int32),
        ],
        scratch_types=[
            pltpu.VMEM((_RCHUNK, _N), jnp.float32),
            pltpu.VMEM((_RCHUNK, _N), jnp.float32),
            pltpu.VMEM((_RCHUNK, _S), jnp.int32),
            pltpu.VMEM((_RCHUNK, _S), jnp.int32),
            pltpu.VMEM((_SEGROWS, _N), jnp.int32),
            pltpu.VMEM((_RCHUNK, _S), jnp.float32),
            pltpu.VMEM((_RCHUNK, _S), jnp.float32),
            pltpu.VMEM((_RCHUNK, _S), jnp.int32),
            pltpu.VMEM((_RCHUNK, _S), jnp.int32),
            pltpu.SemaphoreType.DMA,
            pltpu.SemaphoreType.DMA,
            pltpu.SemaphoreType.DMA,
            pltpu.SemaphoreType.DMA,
        ],
    )(_sc_body)
    return fn(a2, sample_addrs, segtab)


def _loss_body(l_ref, s_ref, seg_ref, out_ref, acc_ref):
    step = pl.program_id(0)
    nsteps = pl.num_programs(0)

    @pl.when(step == 0)
    def _():
        acc_ref[0] = 0.0
        acc_ref[1] = 0.0

    logits = l_ref[...]
    samples = s_ref[...]
    seg = seg_ref[...]
    mask = 1.0 - (seg == 0).astype(jnp.float32)
    targets = (samples == seg).astype(jnp.float32)
    maxes = jnp.max(logits, axis=-1, keepdims=True)
    shifted = logits - maxes
    denom = jnp.sum(jnp.exp(shifted), axis=-1, keepdims=True) * mask + 1e-9
    y_pred_log = jnp.maximum(shifted - jnp.log(denom),
                             jnp.float32(np.log(1e-8)))
    t_sum = jnp.sum(targets, axis=-1, keepdims=True) + 1e-9
    tl_sum = jnp.sum(targets * y_pred_log, axis=-1, keepdims=True)
    t_cnt = jnp.sum(targets, axis=-1, keepdims=True)
    kl_row = (mask / t_sum) * (-jnp.log(t_sum) * t_cnt - tl_sum)
    acc_ref[0] += jnp.sum(kl_row)
    acc_ref[1] += jnp.sum(mask)

    @pl.when(step == nsteps - 1)
    def _():
        out_ref[0, 0] = acc_ref[0]
        out_ref[0, 1] = acc_ref[1]


def _loss_partial(logits_b, samples_b, seg_col_b):
    rb = 512
    grid = (_N // rb,)
    return pl.pallas_call(
        _loss_body,
        grid=grid,
        in_specs=[
            pl.BlockSpec((rb, _S), lambda i: (i, 0)),
            pl.BlockSpec((rb, _S), lambda i: (i, 0)),
            pl.BlockSpec((rb, 1), lambda i: (i, 0)),
        ],
        out_specs=pl.BlockSpec((1, 2), lambda i: (0, 0),
                               memory_space=pltpu.SMEM),
        out_shape=jax.ShapeDtypeStruct((1, 2), jnp.float32),
        scratch_shapes=[pltpu.SMEM((2,), jnp.float32)],
    )(logits_b, samples_b, seg_col_b)


def kernel(features, segment_target, rand_global_inds,
           W_feat, b_feat, W_key, b_key, W_query, b_query):
    x = features.reshape(_B, _IN_DIM, _N)
    wqc = jnp.dot(W_query, W_feat, precision=lax.Precision.HIGHEST)
    wkc = jnp.dot(W_key, W_feat, precision=lax.Precision.HIGHEST)
    qbias = (jnp.dot(W_query, b_feat) + b_query).reshape(1, _C)
    kbias = (jnp.dot(W_key, b_feat) + b_key).reshape(_C, 1)

    q, kt = _projections(x, wqc, wkc, qbias, kbias)
    rand32 = rand_global_inds.astype(jnp.int32)
    seg = segment_target.reshape(_B, _N).astype(jnp.int32)
    local_tab = jnp.asarray(_LOCAL_ADDR)

    logits_halves, partials = [], []
    for b in range(_B):
        a_b = _affinity(q[b:b + 1], kt[b:b + 1]).reshape(_N, _N)
        idx_b = _assemble_inds(local_tab, rand32[b:b + 1])
        segtab_b = jnp.broadcast_to(seg[b].reshape(1, _N), (_SEGROWS, _N))
        logits_b, samples_b = _sc_gather(a_b, idx_b, segtab_b)
        logits_halves.append(logits_b)
        partials.append(_loss_partial(logits_b, samples_b,
                                      seg[b].reshape(_N, 1)))

    psum = partials[0] + partials[1]
    loss = psum[0, 0] / (psum[0, 1] + 1e-9)
    logits = jnp.stack(logits_halves).reshape(_B, _N, _S)
    return (logits, loss)


# speedup vs baseline: 593.2636x; 1.1933x over previous
import functools

import numpy as np
import jax
import jax.numpy as jnp
from jax import lax
from jax.experimental import pallas as pl
from jax.experimental.pallas import tpu as pltpu
from jax.experimental.pallas import tpu_sc as plsc

_B = 2
_IN_DIM = 256
_C = 32
_H = _W = 64
_N = _H * _W
_KWIN = 25
_S = 1024
_NLOC = _KWIN * _KWIN
_NRAND = _S - _NLOC
_ROWS = _B * _N
_INV_SQRT_C = float(_C) ** -0.5

_NUM_WORKERS = 32
_ROWS_PER_W = _N // _NUM_WORKERS
_RCHUNK = 4
_NCHUNK = _ROWS_PER_W // _RCHUNK
_NGROUP = _S // 16
_SEGROWS = _RCHUNK


def _local_window_inds():
    half = (_KWIN - 1) // 2
    idx = np.arange(_N, dtype=np.int64).reshape(_H, _W)
    padded = np.zeros((_H + 2 * half, _W + 2 * half), dtype=np.int64)
    padded[half:half + _H, half:half + _W] = idx
    win = np.lib.stride_tricks.sliding_window_view(padded, (_KWIN, _KWIN))
    return win.reshape(_N, _NLOC).astype(np.int32)


_LOCAL_INDS = _local_window_inds()
_LOCAL_ADDR = (((np.arange(_N, dtype=np.int32)[:, None] % _RCHUNK) << 12)
               + _LOCAL_INDS)


def _proj_body(x_ref, wq_ref, wk_ref, qb_ref, kb_ref, q_ref, kt_ref):
    x = x_ref[0]
    q = lax.dot_general(x, wq_ref[...], (((0,), (1,)), ((), ())),
                        preferred_element_type=jnp.float32,
                        precision=lax.Precision.HIGHEST)
    q_ref[0] = q + qb_ref[...]
    kt = lax.dot_general(wk_ref[...], x, (((1,), (0,)), ((), ())),
                         preferred_element_type=jnp.float32,
                         precision=lax.Precision.HIGHEST)
    kt_ref[0] = kt + kb_ref[...]


def _projections(x, wqc, wkc, qbias, kbias):
    bn = 2048
    grid = (_B, _N // bn)
    return pl.pallas_call(
        _proj_body,
        grid=grid,
        in_specs=[
            pl.BlockSpec((1, _IN_DIM, bn), lambda b, j: (b, 0, j)),
            pl.BlockSpec((_C, _IN_DIM), lambda b, j: (0, 0)),
            pl.BlockSpec((_C, _IN_DIM), lambda b, j: (0, 0)),
            pl.BlockSpec((1, _C), lambda b, j: (0, 0)),
            pl.BlockSpec((_C, 1), lambda b, j: (0, 0)),
        ],
        out_specs=[
            pl.BlockSpec((1, bn, _C), lambda b, j: (b, j, 0)),
            pl.BlockSpec((1, _C, bn), lambda b, j: (b, 0, j)),
        ],
        out_shape=[
            jax.ShapeDtypeStruct((_B, _N, _C), jnp.float32),
            jax.ShapeDtypeStruct((_B, _C, _N), jnp.float32),
        ],
    )(x, wqc, wkc, qbias, kbias)


def _aff_body(q_ref, kt_ref, a_ref):
    a = lax.dot_general(q_ref[0], kt_ref[0], (((1,), (0,)), ((), ())),
                        preferred_element_type=jnp.float32,
                        precision=lax.Precision.DEFAULT)
    a_ref[0] = a * _INV_SQRT_C


def _affinity(qb, ktb):
    bm, bn = 512, 2048
    grid = (_N // bm, _N // bn)
    return pl.pallas_call(
        _aff_body,
        grid=grid,
        in_specs=[
            pl.BlockSpec((1, bm, _C), lambda i, j: (0, i, 0)),
            pl.BlockSpec((1, _C, bn), lambda i, j: (0, 0, j)),
        ],
        out_specs=pl.BlockSpec((1, bm, bn), lambda i, j: (0, i, j)),
        out_shape=jax.ShapeDtypeStruct((1, _N, _N), jnp.float32),
    )(qb, ktb)


def _idx_body(loc_ref, rand_ref, out_ref):
    rb = out_ref.shape[0]
    row = jax.lax.broadcasted_iota(jnp.int32, (rb, _NRAND), 0)
    out_ref[:, 0:_NLOC] = loc_ref[...]
    out_ref[:, _NLOC:_S] = ((row & (_RCHUNK - 1)) << 12) + rand_ref[0]


def _assemble_inds(local_tab, rand_b):
    rb = 2048
    grid = (_N // rb,)
    return pl.pallas_call(
        _idx_body,
        grid=grid,
        in_specs=[
            pl.BlockSpec((rb, _NLOC), lambda i: (i, 0)),
            pl.BlockSpec((1, rb, _NRAND), lambda i: (0, i, 0)),
        ],
        out_specs=pl.BlockSpec((rb, _S), lambda i: (i, 0)),
        out_shape=jax.ShapeDtypeStruct((_N, _S), jnp.int32),
    )(local_tab, rand_b)


def _sc_body(a_hbm, idx_hbm, segtab_hbm, outl_hbm, outs_hbm,
             a_v0, a_v1, idx_v0, idx_v1, segtab_v,
             outl_v0, outl_v1, outs_v0, outs_v1,
             sin0, sin1, sout0, sout1):
    wid = lax.axis_index("s") * 2 + lax.axis_index("c")
    pltpu.sync_copy(segtab_hbm, segtab_v)
    row0 = wid * _ROWS_PER_W

    bufs = ((a_v0, idx_v0, outl_v0, outs_v0, sin0, sout0),
            (a_v1, idx_v1, outl_v1, outs_v1, sin1, sout1))

    def start_in(ci, buf):
        a_v, idx_v, _, _, sin, _ = buf
        base = row0 + ci * _RCHUNK
        pltpu.async_copy(a_hbm.at[pl.ds(base, _RCHUNK)], a_v, sin)
        pltpu.async_copy(idx_hbm.at[pl.ds(base, _RCHUNK)], idx_v, sin)

    def wait_in(ci, buf):
        a_v, idx_v, _, _, sin, _ = buf
        base = row0 + ci * _RCHUNK
        pltpu.make_async_copy(a_hbm.at[pl.ds(base, _RCHUNK)], a_v, sin).wait()
        pltpu.make_async_copy(idx_hbm.at[pl.ds(base, _RCHUNK)], idx_v,
                              sin).wait()

    def start_out(ci, buf):
        _, _, outl_v, outs_v, _, sout = buf
        base = row0 + ci * _RCHUNK
        pltpu.async_copy(outl_v, outl_hbm.at[pl.ds(base, _RCHUNK)], sout)
        pltpu.async_copy(outs_v, outs_hbm.at[pl.ds(base, _RCHUNK)], sout)

    def wait_out(ci, buf):
        _, _, outl_v, outs_v, _, sout = buf
        base = row0 + ci * _RCHUNK
        pltpu.make_async_copy(outl_v, outl_hbm.at[pl.ds(base, _RCHUNK)],
                              sout).wait()
        pltpu.make_async_copy(outs_v, outs_hbm.at[pl.ds(base, _RCHUNK)],
                              sout).wait()

    def compute(buf):
        a_v, idx_v, outl_v, outs_v, _, _ = buf

        def row_body(r, _):
            def grp_body(g):
                sl = pl.ds(g * 16, 16)
                addrs = idx_v[r, sl]
                hi = addrs >> 12
                lo = addrs & 4095
                outl_v[r, sl] = plsc.load_gather(a_v, [hi, lo])
                outs_v[r, sl] = plsc.load_gather(segtab_v, [hi, lo])

            plsc.parallel_loop(0, _NGROUP, 1, unroll=8)(grp_body)
            return 0

        lax.fori_loop(0, _RCHUNK, row_body, 0)

    start_in(0, bufs[0])

    def pair_body(i, _):
        c0 = i * 2
        start_in(c0 + 1, bufs[1])
        wait_in(c0, bufs[0])

        @pl.when(i > 0)
        def _():
            wait_out(c0 - 2, bufs[0])

        compute(bufs[0])
        start_out(c0, bufs[0])

        @pl.when(i < _NCHUNK // 2 - 1)
        def _():
            start_in(c0 + 2, bufs[0])

        wait_in(c0 + 1, bufs[1])

        @pl.when(i > 0)
        def _():
            wait_out(c0 - 1, bufs[1])

        compute(bufs[1])
        start_out(c0 + 1, bufs[1])
        return 0

    lax.fori_loop(0, _NCHUNK // 2, pair_body, 0)
    wait_out(_NCHUNK - 2, bufs[0])
    wait_out(_NCHUNK - 1, bufs[1])


def _sc_gather(a2, sample_addrs, segtab):
    fn = functools.partial(
        pl.kernel,
        mesh=plsc.VectorSubcoreMesh(core_axis_name="c", subcore_axis_name="s"),
        compiler_params=pltpu.CompilerParams(needs_layout_passes=False),
        out_type=[
            jax.ShapeDtypeStruct((_N, _S), jnp.float32),
            jax.ShapeDtypeStruct((_N, _S), jnp.int32),
        ],
        scratch_types=[
            pltpu.VMEM((_RCHUNK, _N), jnp.float32),
            pltpu.VMEM((_RCHUNK, _N), jnp.float32),
            pltpu.VMEM((_RCHUNK, _S), jnp.int32),
            pltpu.VMEM((_RCHUNK, _S), jnp.int32),
            pltpu.VMEM((_SEGROWS, _N), jnp.int32),
            pltpu.VMEM((_RCHUNK, _S), jnp.float32),
            pltpu.VMEM((_RCHUNK, _S), jnp.float32),
            pltpu.VMEM((_RCHUNK, _S), jnp.int32),
            pltpu.VMEM((_RCHUNK, _S), jnp.int32),
            pltpu.SemaphoreType.DMA,
            pltpu.SemaphoreType.DMA,
            pltpu.SemaphoreType.DMA,
            pltpu.SemaphoreType.DMA,
        ],
    )(_sc_body)
    return fn(a2, sample_addrs, segtab)


def _loss_body(l_ref, s_ref, seg_ref, out_ref, acc_ref):
    step = pl.program_id(0)
    nsteps = pl.num_programs(0)

    @pl.when(step == 0)
    def _():
        acc_ref[0] = 0.0
        acc_ref[1] = 0.0

    logits = l_ref[...]
    samples = s_ref[...]
    seg = seg_ref[...]
    mask = 1.0 - (seg == 0).astype(jnp.float32)
    targets = (samples == seg).astype(jnp.float32)
    maxes = jnp.max(logits, axis=-1, keepdims=True)
    shifted = logits - maxes
    denom = jnp.sum(jnp.exp(shifted), axis=-1, keepdims=True) * mask + 1e-9
    y_pred_log = jnp.maximum(shifted - jnp.log(denom),
                             jnp.float32(np.log(1e-8)))
    t_sum = jnp.sum(targets, axis=-1, keepdims=True) + 1e-9
    tl_sum = jnp.sum(targets * y_pred_log, axis=-1, keepdims=True)
    t_cnt = jnp.sum(targets, axis=-1, keepdims=True)
    kl_row = (mask / t_sum) * (-jnp.log(t_sum) * t_cnt - tl_sum)
    acc_ref[0] += jnp.sum(kl_row)
    acc_ref[1] += jnp.sum(mask)

    @pl.when(step == nsteps - 1)
    def _():
        out_ref[0, 0] = acc_ref[0]
        out_ref[0, 1] = acc_ref[1]


def _loss_partial(logits_b, samples_b, seg_col_b):
    rb = 512
    grid = (_N // rb,)
    return pl.pallas_call(
        _loss_body,
        grid=grid,
        in_specs=[
            pl.BlockSpec((rb, _S), lambda i: (i, 0)),
            pl.BlockSpec((rb, _S), lambda i: (i, 0)),
            pl.BlockSpec((rb, 1), lambda i: (i, 0)),
        ],
        out_specs=pl.BlockSpec((1, 2), lambda i: (0, 0),
                               memory_space=pltpu.SMEM),
        out_shape=jax.ShapeDtypeStruct((1, 2), jnp.float32),
        scratch_shapes=[pltpu.SMEM((2,), jnp.float32)],
    )(logits_b, samples_b, seg_col_b)


def kernel(features, segment_target, rand_global_inds,
           W_feat, b_feat, W_key, b_key, W_query, b_query):
    x = features.reshape(_B, _IN_DIM, _N)
    wqc = jnp.dot(W_query, W_feat, precision=lax.Precision.HIGHEST)
    wkc = jnp.dot(W_key, W_feat, precision=lax.Precision.HIGHEST)
    qbias = (jnp.dot(W_query, b_feat) + b_query).reshape(1, _C)
    kbias = (jnp.dot(W_key, b_feat) + b_key).reshape(_C, 1)

    q, kt = _projections(x, wqc, wkc, qbias, kbias)
    rand32 = rand_global_inds.astype(jnp.int32)
    seg = segment_target.reshape(_B, _N).astype(jnp.int32)
    local_tab = jnp.asarray(_LOCAL_ADDR)

    logits_halves, partials = [], []
    for b in range(_B):
        a_b = _affinity(q[b:b + 1], kt[b:b + 1]).reshape(_N, _N)
        idx_b = _assemble_inds(local_tab, rand32[b:b + 1])
        segtab_b = jnp.broadcast_to(seg[b].reshape(1, _N), (_SEGROWS, _N))
        logits_b, samples_b = _sc_gather(a_b, idx_b, segtab_b)
        logits_halves.append(logits_b)
        partials.append(_loss_partial(logits_b, samples_b,
                                      seg[b].reshape(_N, 1)))

    psum = partials[0] + partials[1]
    loss = psum[0, 0] / (psum[0, 1] + 1e-9)
    logits = jnp.stack(logits_halves).reshape(_B, _N, _S)
    return (logits, loss)
